# baseline (speedup 1.0000x reference)
import functools

import jax
import jax.numpy as jnp
from jax import lax
from jax.experimental import pallas as pl
from jax.experimental.pallas import tpu as pltpu
from jax.experimental.pallas import tpu_sc as plsc

_CLASSES = 8192
_N_ROWS = 8192
_EPS = 0.1 / (_CLASSES - 2)
_CONF_COEF = 0.9 - _EPS
_ROWS_PER_BLOCK = 512
_GRID = _N_ROWS // _ROWS_PER_BLOCK

_LANES = 16
_NC, _NS = 2, 16
_NW = _NC * _NS
_B_PER_W = _N_ROWS // _NW
_CHUNKS = _B_PER_W // _LANES
_TBL_COLS = 128
_TBL_ROWS = _N_ROWS * (_CLASSES // _TBL_COLS)


def _sc_gather_body(pred_tbl, tgt_hbm, out_hbm, t_v, idx_v, rows_v, acc_v, sem):
    wid = lax.axis_index("s") * _NC + lax.axis_index("c")
    base = wid * _B_PER_W
    pltpu.sync_copy(tgt_hbm.at[pl.ds(base * 1, _B_PER_W)], t_v)

    iota = lax.iota(jnp.int32, _LANES)
    for c in range(_CHUNKS):
        t_c = t_v[pl.ds(c * _LANES, _LANES)]
        ridx = (base + c * _LANES + iota) * (_CLASSES // _TBL_COLS) + (
            lax.shift_right_logical(t_c, 7))
        idx_v[c // 8, pl.ds((c % 8) * _LANES, _LANES)] = ridx

    cps = [
        pltpu.async_copy(
            pred_tbl.at[idx_v.at[h]],
            rows_v.at[pl.ds(h * 128, 128)], sem)
        for h in range(2)
    ]
    for cp in cps:
        cp.wait()

    acc = jnp.zeros((_LANES,), jnp.float32)
    for c in range(_CHUNKS):
        t_c = t_v[pl.ds(c * _LANES, _LANES)]
        lane = jnp.bitwise_and(t_c, _TBL_COLS - 1)
        val = plsc.load_gather(rows_v, [iota + c * _LANES, lane])
        acc = acc + jnp.where(t_c != 0, val, 0.0)
    acc_v[...] = acc
    pltpu.sync_copy(acc_v, out_hbm.at[wid])


_sc_gather = functools.partial(
    pl.kernel,
    mesh=plsc.VectorSubcoreMesh(core_axis_name="c", subcore_axis_name="s"),
    compiler_params=pltpu.CompilerParams(needs_layout_passes=False),
    out_type=jax.ShapeDtypeStruct((_NW, _LANES), jnp.float32),
    scratch_types=[
        pltpu.VMEM((_B_PER_W,), jnp.int32),
        pltpu.VMEM((2, 128), jnp.int32),
        pltpu.VMEM((_B_PER_W, _TBL_COLS), jnp.float32),
        pltpu.VMEM((_LANES,), jnp.float32),
        pltpu.SemaphoreType.DMA,
    ],
)(_sc_gather_body)


def _tc_kernel(pred_ref, tgt_ref, out_ref):
    i = pl.program_id(0)
    block = pred_ref[...]
    t = tgt_ref[0, 0, :]
    m = jnp.max(block, axis=1)
    s = jnp.sum(jnp.exp(block - m[:, None]), axis=1)
    lse = m + jnp.log(s)
    rowsum = jnp.sum(block, axis=1)
    p0 = block[:, 0]
    u = jnp.where(t != 0, lse - _EPS * (rowsum - p0), 0.0)
    part = jnp.reshape(jnp.sum(u) * (1.0 / _N_ROWS), (1, 1))

    @pl.when(i == 0)
    def _init():
        out_ref[...] = jnp.zeros((1, 1), jnp.float32)

    out_ref[...] += part


def _combine_kernel(a_ref, s_ref, o_ref):
    b = jnp.sum(s_ref[...]) * (_CONF_COEF / _N_ROWS)
    o_ref[...] = a_ref[...] - jnp.reshape(b, (1, 1))


def kernel(pred, target):
    tgt = target.astype(jnp.int32)
    pred_tbl = pred.reshape(_TBL_ROWS, _TBL_COLS)

    pt_parts = _sc_gather(pred_tbl, tgt)

    a = pl.pallas_call(
        _tc_kernel,
        grid=(_GRID,),
        in_specs=[
            pl.BlockSpec((_ROWS_PER_BLOCK, _CLASSES), lambda i: (i, 0)),
            pl.BlockSpec((1, 1, _ROWS_PER_BLOCK), lambda i: (i, 0, 0)),
        ],
        out_specs=pl.BlockSpec((1, 1), lambda i: (0, 0)),
        out_shape=jax.ShapeDtypeStruct((1, 1), jnp.float32),
    )(pred, tgt.reshape(_GRID, 1, _ROWS_PER_BLOCK))

    out = pl.pallas_call(
        _combine_kernel,
        out_shape=jax.ShapeDtypeStruct((1, 1), jnp.float32),
    )(a, pt_parts.reshape(4, 128))
    return out[0, 0]


# speedup vs baseline: 3.8369x; 3.8369x over previous
import functools

import jax
import jax.numpy as jnp
from jax import lax
from jax.experimental import pallas as pl
from jax.experimental.pallas import tpu as pltpu
from jax.experimental.pallas import tpu_sc as plsc

_CLASSES = 8192
_N_ROWS = 8192
_EPS = 0.1 / (_CLASSES - 2)
_CONF_COEF = 0.9 - _EPS
_ROWS_PER_BLOCK = 512
_GRID = _N_ROWS // _ROWS_PER_BLOCK

_LANES = 16
_NC, _NS = 2, 16
_NW = _NC * _NS
_B_PER_W = _N_ROWS // _NW
_CHUNKS = _B_PER_W // _LANES
_TBL_COLS = 128
_TBL_ROWS = _N_ROWS * (_CLASSES // _TBL_COLS)


def _sc_gather_body(pred_tbl, tgt_hbm, out_hbm, t_v, idx_v, rows_v, acc_v, sem):
    wid = lax.axis_index("s") * _NC + lax.axis_index("c")
    base = wid * _B_PER_W
    pltpu.sync_copy(tgt_hbm.at[pl.ds(base * 1, _B_PER_W)], t_v)

    iota = lax.iota(jnp.int32, _LANES)
    for c in range(_CHUNKS):
        t_c = t_v[pl.ds(c * _LANES, _LANES)]
        row = base + c * _LANES + iota
        ridx = (lax.shift_right_logical(row, 3) * (_CLASSES // _TBL_COLS)
                + lax.shift_right_logical(t_c, 7)) * 8 + jnp.bitwise_and(row, 7)
        idx_v[c // 8, pl.ds((c % 8) * _LANES, _LANES)] = ridx

    cps = [
        pltpu.async_copy(
            pred_tbl.at[idx_v.at[h]],
            rows_v.at[pl.ds(h * 128, 128)], sem)
        for h in range(2)
    ]
    for cp in cps:
        cp.wait()

    acc = jnp.zeros((_LANES,), jnp.float32)
    for c in range(_CHUNKS):
        t_c = t_v[pl.ds(c * _LANES, _LANES)]
        lane = jnp.bitwise_and(t_c, _TBL_COLS - 1)
        val = plsc.load_gather(rows_v, [iota + c * _LANES, lane])
        acc = acc + jnp.where(t_c != 0, val, 0.0)
    acc_v[...] = acc
    pltpu.sync_copy(acc_v, out_hbm.at[wid])


_sc_gather = functools.partial(
    pl.kernel,
    mesh=plsc.VectorSubcoreMesh(core_axis_name="c", subcore_axis_name="s"),
    compiler_params=pltpu.CompilerParams(needs_layout_passes=False),
    out_type=jax.ShapeDtypeStruct((_NW, _LANES), jnp.float32),
    scratch_types=[
        pltpu.VMEM((_B_PER_W,), jnp.int32),
        pltpu.VMEM((2, 128), jnp.int32),
        pltpu.VMEM((_B_PER_W, _TBL_COLS), jnp.float32),
        pltpu.VMEM((_LANES,), jnp.float32),
        pltpu.SemaphoreType.DMA,
    ],
)(_sc_gather_body)


def _tc_kernel(pred_ref, tgt_ref, out_ref):
    i = pl.program_id(0)
    block = pred_ref[...]
    t = tgt_ref[0, 0, :]
    m = jnp.max(block, axis=1)
    s = jnp.sum(jnp.exp(block - m[:, None]), axis=1)
    lse = m + jnp.log(s)
    rowsum = jnp.sum(block, axis=1)
    p0 = block[:, 0]
    u = jnp.where(t != 0, lse - _EPS * (rowsum - p0), 0.0)
    part = jnp.reshape(jnp.sum(u) * (1.0 / _N_ROWS), (1, 1))

    @pl.when(i == 0)
    def _init():
        out_ref[...] = jnp.zeros((1, 1), jnp.float32)

    out_ref[...] += part


def _combine_kernel(a_ref, s_ref, o_ref):
    b = jnp.sum(s_ref[...]) * (_CONF_COEF / _N_ROWS)
    o_ref[...] = a_ref[...] - jnp.reshape(b, (1, 1))


def kernel(pred, target):
    tgt = target.astype(jnp.int32)
    pred_tbl = (
        pred.reshape(_N_ROWS // 8, 8, _CLASSES // _TBL_COLS, _TBL_COLS)
        .transpose(0, 2, 1, 3)
        .reshape(_TBL_ROWS, _TBL_COLS)
    )

    pt_parts = _sc_gather(pred_tbl, tgt)

    a = pl.pallas_call(
        _tc_kernel,
        grid=(_GRID,),
        in_specs=[
            pl.BlockSpec((_ROWS_PER_BLOCK, _CLASSES), lambda i: (i, 0)),
            pl.BlockSpec((1, 1, _ROWS_PER_BLOCK), lambda i: (i, 0, 0)),
        ],
        out_specs=pl.BlockSpec((1, 1), lambda i: (0, 0)),
        out_shape=jax.ShapeDtypeStruct((1, 1), jnp.float32),
    )(pred, tgt.reshape(_GRID, 1, _ROWS_PER_BLOCK))

    out = pl.pallas_call(
        _combine_kernel,
        out_shape=jax.ShapeDtypeStruct((1, 1), jnp.float32),
    )(a, pt_parts.reshape(4, 128))
    return out[0, 0]


# speedup vs baseline: 3.8377x; 1.0002x over previous
import functools

import jax
import jax.numpy as jnp
from jax import lax
from jax.experimental import pallas as pl
from jax.experimental.pallas import tpu as pltpu
from jax.experimental.pallas import tpu_sc as plsc

_CLASSES = 8192
_N_ROWS = 8192
_EPS = 0.1 / (_CLASSES - 2)
_CONF_COEF = 0.9 - _EPS
_ROWS_PER_BLOCK = 512
_GRID = _N_ROWS // _ROWS_PER_BLOCK

_LANES = 16
_NC, _NS = 2, 16
_NW = _NC * _NS
_B_PER_W = _N_ROWS // _NW
_CHUNKS = _B_PER_W // _LANES
_TBL_COLS = 128
_TBL_ROWS = _N_ROWS * (_CLASSES // _TBL_COLS)


def _sc_gather_body(pred_tbl, tgt_hbm, out_hbm, t_v, idx_v, rows_v, acc_v, sem):
    wid = lax.axis_index("s") * _NC + lax.axis_index("c")
    base = wid * _B_PER_W
    pltpu.sync_copy(tgt_hbm.at[pl.ds(base * 1, _B_PER_W)], t_v)

    iota = lax.iota(jnp.int32, _LANES)
    for c in range(_CHUNKS):
        t_c = t_v[pl.ds(c * _LANES, _LANES)]
        row = base + c * _LANES + iota
        ridx = (lax.shift_right_logical(row, 3) * (_CLASSES // _TBL_COLS)
                + lax.shift_right_logical(t_c, 7)) * 8 + jnp.bitwise_and(row, 7)
        idx_v[c // 8, pl.ds((c % 8) * _LANES, _LANES)] = ridx

    cps = [
        pltpu.async_copy(
            pred_tbl.at[idx_v.at[h]],
            rows_v.at[pl.ds(h * 128, 128)], sem)
        for h in range(2)
    ]
    for cp in cps:
        cp.wait()

    acc = jnp.zeros((_LANES,), jnp.float32)
    for c in range(_CHUNKS):
        t_c = t_v[pl.ds(c * _LANES, _LANES)]
        lane = jnp.bitwise_and(t_c, _TBL_COLS - 1)
        val = plsc.load_gather(rows_v, [iota + c * _LANES, lane])
        acc = acc + jnp.where(t_c != 0, val, 0.0)
    acc_v[...] = acc
    pltpu.sync_copy(acc_v, out_hbm.at[wid])


_sc_gather = functools.partial(
    pl.kernel,
    mesh=plsc.VectorSubcoreMesh(core_axis_name="c", subcore_axis_name="s"),
    compiler_params=pltpu.CompilerParams(needs_layout_passes=False),
    out_type=jax.ShapeDtypeStruct((_NW, _LANES), jnp.float32),
    scratch_types=[
        pltpu.VMEM((_B_PER_W,), jnp.int32),
        pltpu.VMEM((2, 128), jnp.int32),
        pltpu.VMEM((_B_PER_W, _TBL_COLS), jnp.float32),
        pltpu.VMEM((_LANES,), jnp.float32),
        pltpu.SemaphoreType.DMA,
    ],
)(_sc_gather_body)


def _tc_kernel(pred_ref, tgt_ref, out_ref):
    i = pl.program_id(0)
    block = pred_ref[...]
    t = tgt_ref[0, 0, :]
    m = jnp.max(block, axis=1)
    s = jnp.sum(jnp.exp(block - m[:, None]), axis=1)
    lse = m + jnp.log(s)
    rowsum = jnp.sum(block, axis=1)
    p0 = block[:, 0]
    u = jnp.where(t != 0, lse - _EPS * (rowsum - p0), 0.0)
    part = jnp.reshape(jnp.sum(u) * (1.0 / _N_ROWS), (1, 1))

    @pl.when(i == 0)
    def _init():
        out_ref[...] = jnp.zeros((1, 1), jnp.float32)

    out_ref[...] += part


def _combine_kernel(a_ref, s_ref, o_ref):
    b = jnp.sum(s_ref[...]) * (_CONF_COEF / _N_ROWS)
    o_ref[...] = a_ref[...] - jnp.reshape(b, (1, 1))


def kernel(pred, target):
    tgt = target.astype(jnp.int32)
    pred_tbl = (
        pred.reshape(_N_ROWS // 8, 8, _CLASSES // _TBL_COLS, _TBL_COLS)
        .transpose(0, 2, 1, 3)
        .reshape(_TBL_ROWS, _TBL_COLS)
    )

    a = pl.pallas_call(
        _tc_kernel,
        grid=(_GRID,),
        in_specs=[
            pl.BlockSpec((_ROWS_PER_BLOCK, _CLASSES), lambda i: (i, 0)),
            pl.BlockSpec((1, 1, _ROWS_PER_BLOCK), lambda i: (i, 0, 0)),
        ],
        out_specs=pl.BlockSpec((1, 1), lambda i: (0, 0)),
        out_shape=jax.ShapeDtypeStruct((1, 1), jnp.float32),
    )(pred, tgt.reshape(_GRID, 1, _ROWS_PER_BLOCK))

    pt_parts = _sc_gather(pred_tbl, tgt)

    out = pl.pallas_call(
        _combine_kernel,
        out_shape=jax.ShapeDtypeStruct((1, 1), jnp.float32),
    )(a, pt_parts.reshape(4, 128))
    return out[0, 0]
